# baseline (device time: 53084 ns/iter reference)
import jax
import jax.numpy as jnp
from jax import lax
from jax.experimental import pallas as pl
from jax.experimental.pallas import tpu as pltpu

N_DEV = 4
CAP = 352
PAY = 528


def kernel(x, router_W, route_idx, expert_W, shared_W):
    n_tok, d_model = x.shape
    e_loc, _, d_hid = expert_W.shape

    def body(x_ref, rw_ref, idx_ref, ew_ref, sw_ref, out_ref,
             dt_ref, ps_ref, pr_ref, ys_ref, yr_ref,
             fp_send, fp_recv, ry_send, ry_recv):
        my = lax.axis_index("i")

        barrier_sem = pltpu.get_barrier_semaphore()
        for r in range(1, N_DEV):
            pl.semaphore_signal(
                barrier_sem, inc=1,
                device_id=(lax.rem(my + r, N_DEV),),
                device_id_type=pl.DeviceIdType.MESH,
            )
        pl.semaphore_wait(barrier_sem, N_DEV - 1)

        xv = x_ref[...]
        scores = jnp.dot(xv, rw_ref[...], preferred_element_type=jnp.float32)
        smax = jnp.max(scores, axis=-1, keepdims=True)
        pexp = jnp.exp(scores - smax)
        probs = pexp / jnp.sum(pexp, axis=-1, keepdims=True)
        e_ids = lax.broadcasted_iota(jnp.int32, scores.shape, 1)
        onehot = (idx_ref[...] == e_ids).astype(jnp.float32)
        gate = jnp.sum(probs * onehot, axis=-1, keepdims=True)
        route_f = idx_ref[...].astype(jnp.float32)
        payload = jnp.concatenate(
            [xv, gate, route_f,
             jnp.zeros((n_tok, PAY - d_model - 2), jnp.float32)],
            axis=1).astype(jnp.bfloat16)
        dest = idx_ref[...] // e_loc

        col = lax.broadcasted_iota(jnp.int32, (n_tok, N_DEV), 1)
        peer_of_col = lax.rem(my + col, N_DEV)
        i_all = (dest == peer_of_col).astype(jnp.float32)
        row_i = lax.broadcasted_iota(jnp.int32, (n_tok, n_tok), 0)
        col_i = lax.broadcasted_iota(jnp.int32, (n_tok, n_tok), 1)
        ltri = (col_i < row_i).astype(jnp.bfloat16)
        rank = jnp.dot(ltri, i_all.astype(jnp.bfloat16),
                       preferred_element_type=jnp.float32)

        kio = lax.broadcasted_iota(jnp.int32, (n_tok, CAP), 1)
        contract0 = (((0,), (0,)), ((), ()))

        def build_dispatch(r):
            rank_i = rank[:, r:r + 1].astype(jnp.int32)
            d_t = ((kio == rank_i) & (i_all[:, r:r + 1] > 0.5)
                   ).astype(jnp.bfloat16)
            dt_ref[r] = d_t
            ps_ref[r] = lax.dot_general(
                d_t, payload, contract0,
                preferred_element_type=jnp.float32).astype(jnp.bfloat16)

        fwd = []
        for r in range(1, N_DEV):
            build_dispatch(r)
            peer = lax.rem(my + r, N_DEV)
            rp = pltpu.make_async_remote_copy(
                src_ref=ps_ref.at[r], dst_ref=pr_ref.at[r - 1],
                send_sem=fp_send.at[r - 1], recv_sem=fp_recv.at[r - 1],
                device_id=(peer,), device_id_type=pl.DeviceIdType.MESH)
            rp.start()
            fwd.append(rp)
        build_dispatch(0)

        def expert_apply(pay):
            xin = pay[:, 0:d_model].astype(jnp.float32)
            g_c = pay[:, d_model:d_model + 1].astype(jnp.float32)
            r_c = pay[:, d_model + 1:d_model + 2].astype(jnp.float32)
            acc = None
            for j in range(e_loc):
                e_f = (my * e_loc + j).astype(jnp.float32)
                m = jnp.where(r_c == e_f, g_c, 0.0)
                y = jnp.dot(m * xin, ew_ref[j],
                            preferred_element_type=jnp.float32)
                acc = y if acc is None else acc + y
            return acc.astype(jnp.bfloat16)

        ys_ref[0] = expert_apply(ps_ref[0])
        out_ref[...] = jnp.dot(xv, sw_ref[...],
                               preferred_element_type=jnp.float32)
        out_ref[...] += jnp.dot(dt_ref[0], ys_ref[0],
                                preferred_element_type=jnp.float32)

        rets = []
        for r in range(1, N_DEV):
            fwd[r - 1].wait()
            ys_ref[r] = expert_apply(pr_ref[r - 1])
            ry = pltpu.make_async_remote_copy(
                src_ref=ys_ref.at[r], dst_ref=yr_ref.at[r - 1],
                send_sem=ry_send.at[r - 1], recv_sem=ry_recv.at[r - 1],
                device_id=(lax.rem(my + N_DEV - r, N_DEV),),
                device_id_type=pl.DeviceIdType.MESH)
            ry.start()
            rets.append(ry)

        for r in range(1, N_DEV):
            rets[r - 1].wait()
            out_ref[...] += jnp.dot(dt_ref[r], yr_ref[r - 1],
                                    preferred_element_type=jnp.float32)

    return pl.pallas_call(
        body,
        out_shape=jax.ShapeDtypeStruct((n_tok, d_hid), jnp.float32),
        in_specs=[pl.BlockSpec(memory_space=pltpu.VMEM)] * 5,
        out_specs=pl.BlockSpec(memory_space=pltpu.VMEM),
        scratch_shapes=[
            pltpu.VMEM((N_DEV, n_tok, CAP), jnp.bfloat16),
            pltpu.VMEM((N_DEV, CAP, PAY), jnp.bfloat16),
            pltpu.VMEM((N_DEV - 1, CAP, PAY), jnp.bfloat16),
            pltpu.VMEM((N_DEV, CAP, d_hid), jnp.bfloat16),
            pltpu.VMEM((N_DEV - 1, CAP, d_hid), jnp.bfloat16),
            pltpu.SemaphoreType.DMA((N_DEV - 1,)),
            pltpu.SemaphoreType.DMA((N_DEV - 1,)),
            pltpu.SemaphoreType.DMA((N_DEV - 1,)),
            pltpu.SemaphoreType.DMA((N_DEV - 1,)),
        ],
        compiler_params=pltpu.CompilerParams(collective_id=0),
    )(x, router_W, route_idx, expert_W, shared_W)


# device time: 49061 ns/iter; 1.0820x vs baseline; 1.0820x over previous
import jax
import jax.numpy as jnp
from jax import lax
from jax.experimental import pallas as pl
from jax.experimental.pallas import tpu as pltpu

N_DEV = 4
CAP = 352
PAY = 528


def kernel(x, router_W, route_idx, expert_W, shared_W):
    n_tok, d_model = x.shape
    e_loc, _, d_hid = expert_W.shape

    def body(x_ref, rw_ref, idx_ref, ew_ref, sw_ref, out_ref,
             dt_ref, ps_ref, pr_ref, ys_ref, yr_ref,
             fp_send, fp_recv, ry_send, ry_recv):
        my = lax.axis_index("i")

        barrier_sem = pltpu.get_barrier_semaphore()
        for r in range(1, N_DEV):
            pl.semaphore_signal(
                barrier_sem, inc=1,
                device_id=(lax.rem(my + r, N_DEV),),
                device_id_type=pl.DeviceIdType.MESH,
            )
        pl.semaphore_wait(barrier_sem, N_DEV - 1)

        xv = x_ref[...]
        scores = jnp.dot(xv, rw_ref[...], preferred_element_type=jnp.float32)
        smax = jnp.max(scores, axis=-1, keepdims=True)
        pexp = jnp.exp(scores - smax)
        probs = pexp / jnp.sum(pexp, axis=-1, keepdims=True)
        e_ids = lax.broadcasted_iota(jnp.int32, scores.shape, 1)
        onehot = (idx_ref[...] == e_ids).astype(jnp.float32)
        gate = jnp.sum(probs * onehot, axis=-1, keepdims=True)
        route_f = idx_ref[...].astype(jnp.float32)
        payload = jnp.concatenate(
            [xv, gate, route_f,
             jnp.zeros((n_tok, PAY - d_model - 2), jnp.float32)],
            axis=1).astype(jnp.bfloat16)
        dest = idx_ref[...] // e_loc

        col = lax.broadcasted_iota(jnp.int32, (n_tok, N_DEV), 1)
        peer_of_col = lax.rem(my + col, N_DEV)
        i_all = (dest == peer_of_col).astype(jnp.float32)
        row_i = lax.broadcasted_iota(jnp.int32, (n_tok, n_tok), 0)
        col_i = lax.broadcasted_iota(jnp.int32, (n_tok, n_tok), 1)
        ltri = (col_i < row_i).astype(jnp.bfloat16)
        rank = jnp.dot(ltri, i_all.astype(jnp.bfloat16),
                       preferred_element_type=jnp.float32)

        rank_m = jnp.where(i_all > 0.5, rank, -1.0)
        kio = lax.broadcasted_iota(jnp.int32, (n_tok, CAP), 1)
        kio_r = lax.broadcasted_iota(jnp.int32, (CAP, n_tok), 0)

        def build_dispatch(r):
            rm = rank_m[:, r:r + 1].astype(jnp.int32)
            dt_ref[r] = (kio == rm).astype(jnp.bfloat16)
            rm_row = jnp.reshape(rm, (1, n_tok))
            d_row = (kio_r == rm_row).astype(jnp.bfloat16)
            ps_ref[r] = jnp.dot(
                d_row, payload,
                preferred_element_type=jnp.float32).astype(jnp.bfloat16)

        fwd = []
        for r in range(1, N_DEV):
            build_dispatch(r)
            peer = lax.rem(my + r, N_DEV)
            rp = pltpu.make_async_remote_copy(
                src_ref=ps_ref.at[r], dst_ref=pr_ref.at[r - 1],
                send_sem=fp_send.at[r - 1], recv_sem=fp_recv.at[r - 1],
                device_id=(peer,), device_id_type=pl.DeviceIdType.MESH)
            rp.start()
            fwd.append(rp)
        build_dispatch(0)

        ew_stack = ew_ref[...].reshape(e_loc * d_model, d_hid)

        def expert_apply(pay):
            xin = pay[:, 0:d_model].astype(jnp.float32)
            g_c = pay[:, d_model:d_model + 1].astype(jnp.float32)
            r_c = pay[:, d_model + 1:d_model + 2].astype(jnp.float32)
            xg = g_c * xin
            parts = []
            for j in range(e_loc):
                e_f = (my * e_loc + j).astype(jnp.float32)
                sel = (r_c == e_f).astype(jnp.float32)
                parts.append(sel * xg)
            xcat = jnp.concatenate(parts, axis=1)
            return jnp.dot(xcat, ew_stack,
                           preferred_element_type=jnp.float32
                           ).astype(jnp.bfloat16)

        ys_ref[0] = expert_apply(ps_ref[0])
        out_ref[...] = jnp.dot(xv, sw_ref[...],
                               preferred_element_type=jnp.float32)
        out_ref[...] += jnp.dot(dt_ref[0], ys_ref[0],
                                preferred_element_type=jnp.float32)

        rets = []
        for r in range(1, N_DEV):
            fwd[r - 1].wait()
            ys_ref[r] = expert_apply(pr_ref[r - 1])
            ry = pltpu.make_async_remote_copy(
                src_ref=ys_ref.at[r], dst_ref=yr_ref.at[r - 1],
                send_sem=ry_send.at[r - 1], recv_sem=ry_recv.at[r - 1],
                device_id=(lax.rem(my + N_DEV - r, N_DEV),),
                device_id_type=pl.DeviceIdType.MESH)
            ry.start()
            rets.append(ry)

        for r in range(1, N_DEV):
            rets[r - 1].wait()
            out_ref[...] += jnp.dot(dt_ref[r], yr_ref[r - 1],
                                    preferred_element_type=jnp.float32)

    return pl.pallas_call(
        body,
        out_shape=jax.ShapeDtypeStruct((n_tok, d_hid), jnp.float32),
        in_specs=[pl.BlockSpec(memory_space=pltpu.VMEM)] * 5,
        out_specs=pl.BlockSpec(memory_space=pltpu.VMEM),
        scratch_shapes=[
            pltpu.VMEM((N_DEV, n_tok, CAP), jnp.bfloat16),
            pltpu.VMEM((N_DEV, CAP, PAY), jnp.bfloat16),
            pltpu.VMEM((N_DEV - 1, CAP, PAY), jnp.bfloat16),
            pltpu.VMEM((N_DEV, CAP, d_hid), jnp.bfloat16),
            pltpu.VMEM((N_DEV - 1, CAP, d_hid), jnp.bfloat16),
            pltpu.SemaphoreType.DMA((N_DEV - 1,)),
            pltpu.SemaphoreType.DMA((N_DEV - 1,)),
            pltpu.SemaphoreType.DMA((N_DEV - 1,)),
            pltpu.SemaphoreType.DMA((N_DEV - 1,)),
        ],
        compiler_params=pltpu.CompilerParams(collective_id=0),
    )(x, router_W, route_idx, expert_W, shared_W)


# device time: 49048 ns/iter; 1.0823x vs baseline; 1.0003x over previous
import jax
import jax.numpy as jnp
from jax import lax
from jax.experimental import pallas as pl
from jax.experimental.pallas import tpu as pltpu

N_DEV = 4
CAP = 352
PAY = 528


def kernel(x, router_W, route_idx, expert_W, shared_W):
    n_tok, d_model = x.shape
    e_loc, _, d_hid = expert_W.shape

    def body(x_ref, rw_ref, idx_ref, ew_ref, sw_ref, out_ref,
             dt_ref, ps_ref, pr_ref, ys_ref, yr_ref,
             fp_send, fp_recv, ry_send, ry_recv):
        my = lax.axis_index("i")

        barrier_sem = pltpu.get_barrier_semaphore()
        for r in range(1, N_DEV):
            pl.semaphore_signal(
                barrier_sem, inc=1,
                device_id=(lax.rem(my + r, N_DEV),),
                device_id_type=pl.DeviceIdType.MESH,
            )
        pl.semaphore_wait(barrier_sem, N_DEV - 1)

        xv = x_ref[...]
        scores = jnp.dot(xv, rw_ref[...], preferred_element_type=jnp.float32)
        smax = jnp.max(scores, axis=-1, keepdims=True)
        pexp = jnp.exp(scores - smax)
        probs = pexp / jnp.sum(pexp, axis=-1, keepdims=True)
        e_ids = lax.broadcasted_iota(jnp.int32, scores.shape, 1)
        onehot = (idx_ref[...] == e_ids).astype(jnp.float32)
        gate = jnp.sum(probs * onehot, axis=-1, keepdims=True)
        route_f = idx_ref[...].astype(jnp.float32)
        payload = jnp.concatenate(
            [xv, gate, route_f,
             jnp.zeros((n_tok, PAY - d_model - 2), jnp.float32)],
            axis=1).astype(jnp.bfloat16)
        dest = idx_ref[...] // e_loc

        col = lax.broadcasted_iota(jnp.int32, (n_tok, N_DEV), 1)
        peer_of_col = lax.rem(my + col, N_DEV)
        i_all = (dest == peer_of_col).astype(jnp.float32)
        row_i = lax.broadcasted_iota(jnp.int32, (n_tok, n_tok), 0)
        col_i = lax.broadcasted_iota(jnp.int32, (n_tok, n_tok), 1)
        ltri = (col_i < row_i).astype(jnp.bfloat16)
        rank = jnp.dot(ltri, i_all.astype(jnp.bfloat16),
                       preferred_element_type=jnp.float32)

        rank_m = jnp.where(i_all > 0.5, rank, -1.0)
        kio = lax.broadcasted_iota(jnp.int32, (n_tok, CAP), 1)
        kio_r = lax.broadcasted_iota(jnp.int32, (CAP, n_tok), 0)

        def build_dispatch(r):
            rm = rank_m[:, r:r + 1].astype(jnp.int32)
            dt_ref[r] = (kio == rm).astype(jnp.bfloat16)
            rm_row = jnp.reshape(rm, (1, n_tok))
            d_row = (kio_r == rm_row).astype(jnp.bfloat16)
            ps_ref[r] = jnp.dot(
                d_row, payload,
                preferred_element_type=jnp.float32).astype(jnp.bfloat16)

        fwd = []
        for r in range(1, N_DEV):
            build_dispatch(r)
            peer = lax.rem(my + r, N_DEV)
            rp = pltpu.make_async_remote_copy(
                src_ref=ps_ref.at[r], dst_ref=pr_ref.at[r - 1],
                send_sem=fp_send.at[r - 1], recv_sem=fp_recv.at[r - 1],
                device_id=(peer,), device_id_type=pl.DeviceIdType.MESH)
            rp.start()
            fwd.append(rp)
        build_dispatch(0)

        ew_stack = ew_ref[...].reshape(e_loc * d_model, d_hid)
        h2 = d_hid // 2

        def expert_xcat(pay):
            xin = pay[:, 0:d_model].astype(jnp.float32)
            g_c = pay[:, d_model:d_model + 1].astype(jnp.float32)
            r_c = pay[:, d_model + 1:d_model + 2].astype(jnp.float32)
            xg = g_c * xin
            parts = []
            for j in range(e_loc):
                e_f = (my * e_loc + j).astype(jnp.float32)
                sel = (r_c == e_f).astype(jnp.float32)
                parts.append(sel * xg)
            return jnp.concatenate(parts, axis=1)

        def y_half(xcat, h):
            return jnp.dot(xcat, ew_stack[:, h * h2:(h + 1) * h2],
                           preferred_element_type=jnp.float32
                           ).astype(jnp.bfloat16)

        xcat0 = expert_xcat(ps_ref[0])
        for h in range(2):
            ys_ref[0, h] = y_half(xcat0, h)
        out_ref[...] = jnp.dot(xv, sw_ref[...],
                               preferred_element_type=jnp.float32)
        for h in range(2):
            out_ref[:, h * h2:(h + 1) * h2] += jnp.dot(
                dt_ref[0], ys_ref[0, h],
                preferred_element_type=jnp.float32)

        rets = []
        for r in range(1, N_DEV):
            fwd[r - 1].wait()
            xcat = expert_xcat(pr_ref[r - 1])
            for h in range(2):
                ys_ref[r, h] = y_half(xcat, h)
                ry = pltpu.make_async_remote_copy(
                    src_ref=ys_ref.at[r, h], dst_ref=yr_ref.at[r - 1, h],
                    send_sem=ry_send.at[r - 1, h],
                    recv_sem=ry_recv.at[r - 1, h],
                    device_id=(lax.rem(my + N_DEV - r, N_DEV),),
                    device_id_type=pl.DeviceIdType.MESH)
                ry.start()
                rets.append((r, h, ry))

        for r, h, ry in rets:
            ry.wait()
            out_ref[:, h * h2:(h + 1) * h2] += jnp.dot(
                dt_ref[r], yr_ref[r - 1, h],
                preferred_element_type=jnp.float32)

    return pl.pallas_call(
        body,
        out_shape=jax.ShapeDtypeStruct((n_tok, d_hid), jnp.float32),
        in_specs=[pl.BlockSpec(memory_space=pltpu.VMEM)] * 5,
        out_specs=pl.BlockSpec(memory_space=pltpu.VMEM),
        scratch_shapes=[
            pltpu.VMEM((N_DEV, n_tok, CAP), jnp.bfloat16),
            pltpu.VMEM((N_DEV, CAP, PAY), jnp.bfloat16),
            pltpu.VMEM((N_DEV - 1, CAP, PAY), jnp.bfloat16),
            pltpu.VMEM((N_DEV, 2, CAP, d_hid // 2), jnp.bfloat16),
            pltpu.VMEM((N_DEV - 1, 2, CAP, d_hid // 2), jnp.bfloat16),
            pltpu.SemaphoreType.DMA((N_DEV - 1,)),
            pltpu.SemaphoreType.DMA((N_DEV - 1,)),
            pltpu.SemaphoreType.DMA((N_DEV - 1, 2)),
            pltpu.SemaphoreType.DMA((N_DEV - 1, 2)),
        ],
        compiler_params=pltpu.CompilerParams(collective_id=0),
    )(x, router_W, route_idx, expert_W, shared_W)


# device time: 48373 ns/iter; 1.0974x vs baseline; 1.0140x over previous
import jax
import jax.numpy as jnp
from jax import lax
from jax.experimental import pallas as pl
from jax.experimental.pallas import tpu as pltpu

N_DEV = 4
CAP = 352
PAY = 528


def kernel(x, router_W, route_idx, expert_W, shared_W):
    n_tok, d_model = x.shape
    e_loc, _, d_hid = expert_W.shape

    def body(x_ref, rw_ref, idx_ref, ew_ref, sw_ref, out_ref,
             dt_ref, ps_ref, pr_ref, ys_ref, yr_ref,
             fp_send, fp_recv, ry_send, ry_recv):
        my = lax.axis_index("i")

        barrier_sem = pltpu.get_barrier_semaphore()
        for r in range(1, N_DEV):
            pl.semaphore_signal(
                barrier_sem, inc=1,
                device_id=(lax.rem(my + r, N_DEV),),
                device_id_type=pl.DeviceIdType.MESH,
            )

        xv = x_ref[...]
        scores = jnp.dot(xv, rw_ref[...], preferred_element_type=jnp.float32)
        smax = jnp.max(scores, axis=-1, keepdims=True)
        pexp = jnp.exp(scores - smax)
        probs = pexp / jnp.sum(pexp, axis=-1, keepdims=True)
        e_ids = lax.broadcasted_iota(jnp.int32, scores.shape, 1)
        onehot = (idx_ref[...] == e_ids).astype(jnp.float32)
        gate = jnp.sum(probs * onehot, axis=-1, keepdims=True)
        route_f = idx_ref[...].astype(jnp.float32)
        payload = jnp.concatenate(
            [xv, gate, route_f,
             jnp.zeros((n_tok, PAY - d_model - 2), jnp.float32)],
            axis=1).astype(jnp.bfloat16)
        dest = idx_ref[...] // e_loc

        col = lax.broadcasted_iota(jnp.int32, (n_tok, N_DEV), 1)
        peer_of_col = lax.rem(my + col, N_DEV)
        i_all = (dest == peer_of_col).astype(jnp.float32)
        t_b = 256
        n_blk = n_tok // t_b
        row_i = lax.broadcasted_iota(jnp.int32, (t_b, t_b), 0)
        col_i = lax.broadcasted_iota(jnp.int32, (t_b, t_b), 1)
        ltri = (col_i < row_i).astype(jnp.bfloat16)
        blocks = []
        off = jnp.zeros((1, N_DEV), jnp.float32)
        for b in range(n_blk):
            ib = i_all[b * t_b:(b + 1) * t_b, :]
            rb = jnp.dot(ltri, ib.astype(jnp.bfloat16),
                         preferred_element_type=jnp.float32)
            blocks.append(rb + off)
            off = off + jnp.sum(ib, axis=0, keepdims=True)
        rank = jnp.concatenate(blocks, axis=0)

        rank_m = jnp.where(i_all > 0.5, rank, -1.0)
        kio = lax.broadcasted_iota(jnp.int32, (n_tok, CAP), 1)
        kio_r = lax.broadcasted_iota(jnp.int32, (CAP, n_tok), 0)

        def build_dispatch(r):
            rm = rank_m[:, r:r + 1].astype(jnp.int32)
            dt_ref[r] = (kio == rm).astype(jnp.bfloat16)
            rm_row = jnp.reshape(rm, (1, n_tok))
            d_row = (kio_r == rm_row).astype(jnp.bfloat16)
            ps_ref[r] = jnp.dot(
                d_row, payload,
                preferred_element_type=jnp.float32).astype(jnp.bfloat16)

        fwd = []
        for r in range(1, N_DEV):
            build_dispatch(r)
            if r == 1:
                pl.semaphore_wait(barrier_sem, N_DEV - 1)
            peer = lax.rem(my + r, N_DEV)
            rp = pltpu.make_async_remote_copy(
                src_ref=ps_ref.at[r], dst_ref=pr_ref.at[r - 1],
                send_sem=fp_send.at[r - 1], recv_sem=fp_recv.at[r - 1],
                device_id=(peer,), device_id_type=pl.DeviceIdType.MESH)
            rp.start()
            fwd.append(rp)
        build_dispatch(0)

        ew_stack = ew_ref[...].reshape(e_loc * d_model, d_hid)
        h2 = d_hid // 2

        def expert_xcat(pay):
            xin = pay[:, 0:d_model].astype(jnp.float32)
            g_c = pay[:, d_model:d_model + 1].astype(jnp.float32)
            r_c = pay[:, d_model + 1:d_model + 2].astype(jnp.float32)
            xg = g_c * xin
            parts = []
            for j in range(e_loc):
                e_f = (my * e_loc + j).astype(jnp.float32)
                sel = (r_c == e_f).astype(jnp.float32)
                parts.append(sel * xg)
            return jnp.concatenate(parts, axis=1)

        def y_half(xcat, h):
            return jnp.dot(xcat, ew_stack[:, h * h2:(h + 1) * h2],
                           preferred_element_type=jnp.float32
                           ).astype(jnp.bfloat16)

        xcat0 = expert_xcat(ps_ref[0])
        for h in range(2):
            ys_ref[0, h] = y_half(xcat0, h)
        out_ref[...] = jnp.dot(xv, sw_ref[...],
                               preferred_element_type=jnp.float32)
        for h in range(2):
            out_ref[:, h * h2:(h + 1) * h2] += jnp.dot(
                dt_ref[0], ys_ref[0, h],
                preferred_element_type=jnp.float32)

        rets = []
        for r in range(1, N_DEV):
            fwd[r - 1].wait()
            xcat = expert_xcat(pr_ref[r - 1])
            for h in range(2):
                ys_ref[r, h] = y_half(xcat, h)
                ry = pltpu.make_async_remote_copy(
                    src_ref=ys_ref.at[r, h], dst_ref=yr_ref.at[r - 1, h],
                    send_sem=ry_send.at[r - 1, h],
                    recv_sem=ry_recv.at[r - 1, h],
                    device_id=(lax.rem(my + N_DEV - r, N_DEV),),
                    device_id_type=pl.DeviceIdType.MESH)
                ry.start()
                rets.append((r, h, ry))

        for r, h, ry in rets:
            ry.wait()
            out_ref[:, h * h2:(h + 1) * h2] += jnp.dot(
                dt_ref[r], yr_ref[r - 1, h],
                preferred_element_type=jnp.float32)

    return pl.pallas_call(
        body,
        out_shape=jax.ShapeDtypeStruct((n_tok, d_hid), jnp.float32),
        in_specs=[pl.BlockSpec(memory_space=pltpu.VMEM)] * 5,
        out_specs=pl.BlockSpec(memory_space=pltpu.VMEM),
        scratch_shapes=[
            pltpu.VMEM((N_DEV, n_tok, CAP), jnp.bfloat16),
            pltpu.VMEM((N_DEV, CAP, PAY), jnp.bfloat16),
            pltpu.VMEM((N_DEV - 1, CAP, PAY), jnp.bfloat16),
            pltpu.VMEM((N_DEV, 2, CAP, d_hid // 2), jnp.bfloat16),
            pltpu.VMEM((N_DEV - 1, 2, CAP, d_hid // 2), jnp.bfloat16),
            pltpu.SemaphoreType.DMA((N_DEV - 1,)),
            pltpu.SemaphoreType.DMA((N_DEV - 1,)),
            pltpu.SemaphoreType.DMA((N_DEV - 1, 2)),
            pltpu.SemaphoreType.DMA((N_DEV - 1, 2)),
        ],
        compiler_params=pltpu.CompilerParams(collective_id=0),
    )(x, router_W, route_idx, expert_W, shared_W)


# device time: 45754 ns/iter; 1.1602x vs baseline; 1.0572x over previous
import jax
import jax.numpy as jnp
from jax import lax
from jax.experimental import pallas as pl
from jax.experimental.pallas import tpu as pltpu

N_DEV = 4
CAP = 352
C1 = 288
PAY = 528
HDR = 514


def kernel(x, router_W, route_idx, expert_W, shared_W):
    n_tok, d_model = x.shape
    e_loc, _, d_hid = expert_W.shape

    def body(x_ref, rw_ref, idx_ref, ew_ref, sw_ref, out_ref,
             dt_ref, ps_ref, pr_ref, ys_ref, yr_ref,
             fp1_send, fp1_recv, fp2_send, fp2_recv,
             ry1_send, ry1_recv, ry2_send, ry2_recv):
        my = lax.axis_index("i")

        for r in range(1, N_DEV):
            pr_ref[r - 1, C1:CAP, :] = jnp.zeros(
                (CAP - C1, PAY), jnp.bfloat16)
            yr_ref[r - 1, C1:CAP, :] = jnp.zeros(
                (CAP - C1, d_hid), jnp.bfloat16)

        barrier_sem = pltpu.get_barrier_semaphore()
        for r in range(1, N_DEV):
            pl.semaphore_signal(
                barrier_sem, inc=1,
                device_id=(lax.rem(my + r, N_DEV),),
                device_id_type=pl.DeviceIdType.MESH,
            )

        xv = x_ref[...]
        scores = jnp.dot(xv, rw_ref[...], preferred_element_type=jnp.float32)
        smax = jnp.max(scores, axis=-1, keepdims=True)
        pexp = jnp.exp(scores - smax)
        probs = pexp / jnp.sum(pexp, axis=-1, keepdims=True)
        e_ids = lax.broadcasted_iota(jnp.int32, scores.shape, 1)
        onehot = (idx_ref[...] == e_ids).astype(jnp.float32)
        gate = jnp.sum(probs * onehot, axis=-1, keepdims=True)
        route_f = idx_ref[...].astype(jnp.float32)
        payload = jnp.concatenate(
            [xv, gate, route_f,
             jnp.zeros((n_tok, PAY - d_model - 2), jnp.float32)],
            axis=1).astype(jnp.bfloat16)
        dest = idx_ref[...] // e_loc

        col = lax.broadcasted_iota(jnp.int32, (n_tok, N_DEV), 1)
        peer_of_col = lax.rem(my + col, N_DEV)
        i_all = (dest == peer_of_col).astype(jnp.float32)
        t_b = 256
        row_i = lax.broadcasted_iota(jnp.int32, (t_b, t_b), 0)
        col_i = lax.broadcasted_iota(jnp.int32, (t_b, t_b), 1)
        ltri = (col_i < row_i).astype(jnp.bfloat16)
        blocks = []
        off = jnp.zeros((1, N_DEV), jnp.float32)
        for b in range(n_tok // t_b):
            ib = i_all[b * t_b:(b + 1) * t_b, :]
            rb = jnp.dot(ltri, ib.astype(jnp.bfloat16),
                         preferred_element_type=jnp.float32)
            blocks.append(rb + off)
            off = off + jnp.sum(ib, axis=0, keepdims=True)
        rank = jnp.concatenate(blocks, axis=0)
        counts_b16 = off.astype(jnp.bfloat16)

        rank_m = jnp.where(i_all > 0.5, rank, -1.0)
        kio = lax.broadcasted_iota(jnp.int32, (n_tok, CAP), 1)
        kio_r = lax.broadcasted_iota(jnp.int32, (CAP, n_tok), 0)

        def build_dispatch(r):
            rm = rank_m[:, r:r + 1].astype(jnp.int32)
            dt_ref[r] = (kio == rm).astype(jnp.bfloat16)
            rm_row = jnp.reshape(rm, (1, n_tok))
            d_row = (kio_r == rm_row).astype(jnp.bfloat16)
            ps_ref[r] = jnp.dot(
                d_row, payload,
                preferred_element_type=jnp.float32).astype(jnp.bfloat16)

        thresh = jnp.float32(C1 - 1)
        dec_s = [None] * N_DEV

        fwd1, fwd2 = [], []
        for r in range(1, N_DEV):
            build_dispatch(r)
            cnt_b = counts_b16[0:1, r:r + 1]
            ps_ref[r, 0:1, HDR:HDR + 1] = cnt_b
            dec_s[r] = cnt_b.astype(jnp.float32)[0, 0] > thresh
            if r == 1:
                pl.semaphore_wait(barrier_sem, N_DEV - 1)
            peer = lax.rem(my + r, N_DEV)
            rp1 = pltpu.make_async_remote_copy(
                src_ref=ps_ref.at[r, 0:C1], dst_ref=pr_ref.at[r - 1, 0:C1],
                send_sem=fp1_send.at[r - 1], recv_sem=fp1_recv.at[r - 1],
                device_id=(peer,), device_id_type=pl.DeviceIdType.MESH)
            rp2 = pltpu.make_async_remote_copy(
                src_ref=ps_ref.at[r, C1:CAP], dst_ref=pr_ref.at[r - 1, C1:CAP],
                send_sem=fp2_send.at[r - 1], recv_sem=fp2_recv.at[r - 1],
                device_id=(peer,), device_id_type=pl.DeviceIdType.MESH)
            rp1.start()

            @pl.when(dec_s[r])
            def _():
                rp2.start()

            fwd1.append(rp1)
            fwd2.append(rp2)
        build_dispatch(0)

        ew_stack = ew_ref[...].reshape(e_loc * d_model, d_hid)

        def expert_apply(pay):
            xin = pay[:, 0:d_model].astype(jnp.float32)
            g_c = pay[:, d_model:d_model + 1].astype(jnp.float32)
            r_c = pay[:, d_model + 1:d_model + 2].astype(jnp.float32)
            xg = g_c * xin
            parts = []
            for j in range(e_loc):
                e_f = (my * e_loc + j).astype(jnp.float32)
                sel = (r_c == e_f).astype(jnp.float32)
                parts.append(sel * xg)
            xcat = jnp.concatenate(parts, axis=1)
            return jnp.dot(xcat, ew_stack,
                           preferred_element_type=jnp.float32
                           ).astype(jnp.bfloat16)

        ys_ref[0] = expert_apply(ps_ref[0])
        out_ref[...] = jnp.dot(xv, sw_ref[...],
                               preferred_element_type=jnp.float32)
        out_ref[...] += jnp.dot(dt_ref[0], ys_ref[0],
                                preferred_element_type=jnp.float32)

        ret1, ret2, dec_r = [], [], [None] * N_DEV
        for r in range(1, N_DEV):
            fwd1[r - 1].wait()

            @pl.when(dec_s[r])
            def _():
                fwd2[r - 1].wait_send()

            dec_r[r] = (pr_ref[r - 1, 0:1, HDR:HDR + 1]
                        .astype(jnp.float32)[0, 0] > thresh)

            @pl.when(dec_r[r])
            def _():
                fwd2[r - 1].wait_recv()

            ys_ref[r] = expert_apply(pr_ref[r - 1])
            back = lax.rem(my + N_DEV - r, N_DEV)
            ry1 = pltpu.make_async_remote_copy(
                src_ref=ys_ref.at[r, 0:C1], dst_ref=yr_ref.at[r - 1, 0:C1],
                send_sem=ry1_send.at[r - 1], recv_sem=ry1_recv.at[r - 1],
                device_id=(back,), device_id_type=pl.DeviceIdType.MESH)
            ry2 = pltpu.make_async_remote_copy(
                src_ref=ys_ref.at[r, C1:CAP], dst_ref=yr_ref.at[r - 1, C1:CAP],
                send_sem=ry2_send.at[r - 1], recv_sem=ry2_recv.at[r - 1],
                device_id=(back,), device_id_type=pl.DeviceIdType.MESH)
            ry1.start()

            @pl.when(dec_r[r])
            def _():
                ry2.start()

            ret1.append(ry1)
            ret2.append(ry2)

        for r in range(1, N_DEV):
            ret1[r - 1].wait()

            @pl.when(dec_r[r])
            def _():
                ret2[r - 1].wait_send()

            @pl.when(dec_s[r])
            def _():
                ret2[r - 1].wait_recv()

            out_ref[...] += jnp.dot(dt_ref[r], yr_ref[r - 1],
                                    preferred_element_type=jnp.float32)

    return pl.pallas_call(
        body,
        out_shape=jax.ShapeDtypeStruct((n_tok, d_hid), jnp.float32),
        in_specs=[pl.BlockSpec(memory_space=pltpu.VMEM)] * 5,
        out_specs=pl.BlockSpec(memory_space=pltpu.VMEM),
        scratch_shapes=[
            pltpu.VMEM((N_DEV, n_tok, CAP), jnp.bfloat16),
            pltpu.VMEM((N_DEV, CAP, PAY), jnp.bfloat16),
            pltpu.VMEM((N_DEV - 1, CAP, PAY), jnp.bfloat16),
            pltpu.VMEM((N_DEV, CAP, d_hid), jnp.bfloat16),
            pltpu.VMEM((N_DEV - 1, CAP, d_hid), jnp.bfloat16),
            pltpu.SemaphoreType.DMA((N_DEV - 1,)),
            pltpu.SemaphoreType.DMA((N_DEV - 1,)),
            pltpu.SemaphoreType.DMA((N_DEV - 1,)),
            pltpu.SemaphoreType.DMA((N_DEV - 1,)),
            pltpu.SemaphoreType.DMA((N_DEV - 1,)),
            pltpu.SemaphoreType.DMA((N_DEV - 1,)),
            pltpu.SemaphoreType.DMA((N_DEV - 1,)),
            pltpu.SemaphoreType.DMA((N_DEV - 1,)),
        ],
        compiler_params=pltpu.CompilerParams(collective_id=0),
    )(x, router_W, route_idx, expert_W, shared_W)
